# baseline (speedup 1.0000x reference)
import functools

import jax
import jax.numpy as jnp
from jax import lax
from jax.experimental import pallas as pl
from jax.experimental.pallas import tpu as pltpu
from jax.experimental.pallas import tpu_sc as plsc

_TOKENS = 16384
_D = 256
_CB = 8192
_BT = 512
_BK = 1024
_NT = _TOKENS // _BT
_NK = _CB // _BK

_NC = 2
_NS = 16
_NW = _NC * _NS
_RPW = _TOKENS // _NW
_CH = 128
_NCH = _RPW // _CH


def _dist_argmin_body(x_ref, e_ref, o_ref, bv_ref, bi_ref):
    k = pl.program_id(1)
    x = x_ref[...]
    e = e_ref[...]
    p3 = 2.0 * lax.dot_general(
        x.astype(jnp.bfloat16), e.astype(jnp.bfloat16),
        (((1,), (1,)), ((), ())), preferred_element_type=jnp.float32)
    p1 = jnp.sum(x * x, axis=1, keepdims=True)
    p2 = jnp.sum(e * e, axis=1)
    d = (p1 + p2[None, :]) - p3
    lm = jnp.min(d, axis=1, keepdims=True)
    ii = lax.broadcasted_iota(jnp.int32, d.shape, 1) + k * _BK
    li = jnp.min(jnp.where(d == lm, ii, jnp.int32(2**30)), axis=1, keepdims=True)

    @pl.when(k == 0)
    def _():
        bv_ref[...] = lm
        bi_ref[...] = li

    @pl.when(k > 0)
    def _():
        pv = bv_ref[...]
        pi = bi_ref[...]
        upd = lm < pv
        bv_ref[...] = jnp.where(upd, lm, pv)
        bi_ref[...] = jnp.where(upd, li, pi)

    o_ref[...] = bi_ref[...]


def _dist_argmin(xf, emb, interpret=False):
    return pl.pallas_call(
        _dist_argmin_body,
        grid=(_NT, _NK),
        in_specs=[
            pl.BlockSpec((_BT, _D), lambda t, k: (t, 0)),
            pl.BlockSpec((_BK, _D), lambda t, k: (k, 0)),
        ],
        out_specs=pl.BlockSpec((_BT, 1), lambda t, k: (t, 0)),
        out_shape=jax.ShapeDtypeStruct((_TOKENS, 1), jnp.int32),
        scratch_shapes=[
            pltpu.VMEM((_BT, 1), jnp.float32),
            pltpu.VMEM((_BT, 1), jnp.int32),
        ],
        interpret=interpret,
    )(xf, emb)


def _sc_gather(emb, idx2d):
    mesh = plsc.VectorSubcoreMesh(core_axis_name="c", subcore_axis_name="s")

    @functools.partial(
        pl.kernel,
        mesh=mesh,
        out_type=jax.ShapeDtypeStruct((_TOKENS, _D), jnp.float32),
        scratch_types=[
            pltpu.VMEM((_NCH, _CH), jnp.int32),
            pltpu.VMEM((_CH, _D), jnp.float32),
            pltpu.SemaphoreType.DMA,
        ],
    )
    def gather_k(emb_hbm, idx_hbm, out_hbm, idx_v, rows_v, sem):
        wid = lax.axis_index("s") * _NC + lax.axis_index("c")
        base_chunk = wid * _NCH
        pltpu.sync_copy(idx_hbm.at[pl.ds(base_chunk, _NCH)], idx_v)
        for j in range(_NCH):
            pltpu.async_copy(emb_hbm.at[idx_v.at[j]], rows_v, sem).wait()
            pltpu.sync_copy(rows_v, out_hbm.at[pl.ds((base_chunk + j) * _CH, _CH)])

    return gather_k(emb, idx2d)


def kernel(x, emb):
    b = x.shape[0]
    xf = x.reshape(-1, x.shape[-1])
    idx = _dist_argmin(xf, emb)[:, 0]
    q = _sc_gather(emb, idx.reshape(-1, _CH))
    return q.reshape(b, -1, emb.shape[1]), idx.reshape(b, -1)
